# Initial kernel scaffold; baseline (speedup 1.0000x reference)
import jax
import jax.numpy as jnp
from jax.experimental import pallas as pl


def kernel(pts):
    raise NotImplementedError("write your pallas kernel here")


# speedup vs baseline: 127.2297x; 127.2297x over previous
import functools

import jax
import jax.numpy as jnp
from jax import lax
from jax.experimental import pallas as pl
from jax.experimental.pallas import tpu as pltpu
from jax.experimental.pallas import tpu_sc as plsc

B = 8
N = 262144
IMG = 200
CAM_H = 0.65
NSEG = IMG * IMG
NRANGE = NSEG // 4
CHUNK = 4096
TC_BLK = 65536


def _proj_body(t_ref, pix_ref, d_ref):
    f = 1.0 / jnp.tan(jnp.deg2rad(jnp.float32(45.0)) / 2.0)
    x = t_ref[0]
    y = t_ref[1]
    z = t_ref[2]
    depth = CAM_H - y
    dok = depth > 1e-4
    safe = jnp.where(dok, depth, 1.0)
    ndc_x = x * f / safe
    ndc_y = z * f / safe
    px = jnp.floor((ndc_x + 1.0) * 0.5 * IMG).astype(jnp.int32)
    py = jnp.floor((ndc_y + 1.0) * 0.5 * IMG).astype(jnp.int32)
    valid = dok & (px >= 0) & (px < IMG) & (py >= 0) & (py < IMG)
    pix_ref[...] = jnp.where(valid, py * IMG + px, NSEG)
    d_ref[...] = jnp.where(valid, depth, jnp.inf).astype(jnp.float32)


def _project(pts_t):
    grid = (B * N // TC_BLK,)
    return pl.pallas_call(
        _proj_body,
        grid=grid,
        in_specs=[pl.BlockSpec((3, TC_BLK), lambda i: (0, i))],
        out_specs=[
            pl.BlockSpec((TC_BLK,), lambda i: (i,)),
            pl.BlockSpec((TC_BLK,), lambda i: (i,)),
        ],
        out_shape=[
            jax.ShapeDtypeStruct((B * N,), jnp.int32),
            jax.ShapeDtypeStruct((B * N,), jnp.float32),
        ],
    )(pts_t)


def _sc_vis_body(pix_hbm, d_hbm, out_hbm, pixc, dc, table, thresh, visc,
                 shared):
    c = lax.axis_index("c")
    s = lax.axis_index("s")
    b = c * 4 + s // 4
    r = s % 4
    rlo = r * NRANGE
    bslot = s // 4
    inf = jnp.float32(jnp.inf)

    def _init(i, carry):
        table[pl.ds(i * 16, 16)] = jnp.full((16,), inf, jnp.float32)
        return carry

    lax.fori_loop(0, 5 * NRANGE // 16, _init, 0)

    def _chunk1(ci, carry):
        base = ci * CHUNK
        pltpu.sync_copy(pix_hbm.at[b, pl.ds(base, CHUNK)], pixc)
        pltpu.sync_copy(d_hbm.at[b, pl.ds(base, CHUNK)], dc)

        def _vec(v, carry2):
            p = pixc[pl.ds(v * 16, 16)]
            dv = dc[pl.ds(v * 16, 16)]
            u = p - rlo
            act0 = (u >= 0) & (u < NRANGE)
            us = jnp.where(act0, u, 0)
            occ = plsc.scan_count(us, mask=act0)[0].astype(jnp.int32)
            nmax = jnp.max(jnp.where(act0, occ, -1))

            def _round(i, carry3):
                sel = act0 & (occ == i)
                m0 = plsc.load_gather(table, [us])
                m1 = plsc.load_gather(table, [us + NRANGE])
                m2 = plsc.load_gather(table, [us + 2 * NRANGE])
                m3 = plsc.load_gather(table, [us + 3 * NRANGE])
                m4 = plsc.load_gather(table, [us + 4 * NRANGE])
                dup = ((dv == m0) | (dv == m1) | (dv == m2) | (dv == m3)
                       | (dv == m4))
                ins = sel & (~dup) & (dv < m4)
                t = dv
                n0 = jnp.minimum(m0, t)
                t = jnp.maximum(m0, t)
                n1 = jnp.minimum(m1, t)
                t = jnp.maximum(m1, t)
                n2 = jnp.minimum(m2, t)
                t = jnp.maximum(m2, t)
                n3 = jnp.minimum(m3, t)
                t = jnp.maximum(m3, t)
                n4 = jnp.minimum(m4, t)
                plsc.store_scatter(table, [us], n0, mask=ins)
                plsc.store_scatter(table, [us + NRANGE], n1, mask=ins)
                plsc.store_scatter(table, [us + 2 * NRANGE], n2, mask=ins)
                plsc.store_scatter(table, [us + 3 * NRANGE], n3, mask=ins)
                plsc.store_scatter(table, [us + 4 * NRANGE], n4, mask=ins)
                return carry3

            lax.fori_loop(0, nmax + 1, _round, 0)
            return carry2

        lax.fori_loop(0, CHUNK // 16, _vec, 0)
        return carry

    lax.fori_loop(0, N // CHUNK, _chunk1, 0)

    plsc.subcore_barrier()
    pltpu.sync_copy(table.at[pl.ds(4 * NRANGE, NRANGE)],
                    shared.at[pl.ds(bslot * NSEG + rlo, NRANGE)])
    plsc.subcore_barrier()
    pltpu.sync_copy(shared.at[pl.ds(bslot * NSEG, NSEG)],
                    thresh.at[pl.ds(0, NSEG)])
    thresh[pl.ds(NSEG, 16)] = jnp.full((16,), -inf, jnp.float32)

    def _chunk3(ci, carry):
        base = r * (N // 4) + ci * CHUNK
        pltpu.sync_copy(pix_hbm.at[b, pl.ds(base, CHUNK)], pixc)
        pltpu.sync_copy(d_hbm.at[b, pl.ds(base, CHUNK)], dc)

        def _vec(v, carry2):
            p = pixc[pl.ds(v * 16, 16)]
            dv = dc[pl.ds(v * 16, 16)]
            thr = plsc.load_gather(thresh, [p])
            vis = jnp.where(dv <= thr, jnp.float32(1.0), jnp.float32(0.0))
            visc[pl.ds(v * 16, 16)] = vis
            return carry2

        lax.fori_loop(0, CHUNK // 16, _vec, 0)
        pltpu.sync_copy(visc, out_hbm.at[b, pl.ds(base, CHUNK)])
        return carry

    lax.fori_loop(0, (N // 4) // CHUNK, _chunk3, 0)


_sc_vis = functools.partial(
    pl.kernel,
    mesh=plsc.VectorSubcoreMesh(core_axis_name="c", subcore_axis_name="s"),
    out_type=jax.ShapeDtypeStruct((B, N), jnp.float32),
    compiler_params=pltpu.CompilerParams(needs_layout_passes=False),
    scratch_types=[
        pltpu.VMEM((CHUNK,), jnp.int32),
        pltpu.VMEM((CHUNK,), jnp.float32),
        pltpu.VMEM((5 * NRANGE,), jnp.float32),
        pltpu.VMEM((NSEG + 16,), jnp.float32),
        pltpu.VMEM((CHUNK,), jnp.float32),
        pltpu.VMEM_SHARED((4 * NSEG,), jnp.float32),
    ],
)(_sc_vis_body)


def kernel(pts):
    pts_t = jnp.transpose(pts, (2, 0, 1)).reshape(3, B * N)
    pix, d = _project(pts_t)
    return _sc_vis(pix.reshape(B, N), d.reshape(B, N))


# speedup vs baseline: 137.8204x; 1.0832x over previous
import functools

import jax
import jax.numpy as jnp
from jax import lax
from jax.experimental import pallas as pl
from jax.experimental.pallas import tpu as pltpu
from jax.experimental.pallas import tpu_sc as plsc

B = 8
N = 262144
IMG = 200
CAM_H = 0.65
NSEG = IMG * IMG
NRANGE = NSEG // 4
CHUNK = 4096
TC_BLK = 65536


def _proj_body(t_ref, pix_ref, d_ref):
    f = 1.0 / jnp.tan(jnp.deg2rad(jnp.float32(45.0)) / 2.0)
    x = t_ref[0]
    y = t_ref[1]
    z = t_ref[2]
    depth = CAM_H - y
    dok = depth > 1e-4
    safe = jnp.where(dok, depth, 1.0)
    ndc_x = x * f / safe
    ndc_y = z * f / safe
    px = jnp.floor((ndc_x + 1.0) * 0.5 * IMG).astype(jnp.int32)
    py = jnp.floor((ndc_y + 1.0) * 0.5 * IMG).astype(jnp.int32)
    valid = dok & (px >= 0) & (px < IMG) & (py >= 0) & (py < IMG)
    pix_ref[...] = jnp.where(valid, py * IMG + px, NSEG)
    d_ref[...] = jnp.where(valid, depth, jnp.inf).astype(jnp.float32)


def _project(pts_t):
    grid = (B * N // TC_BLK,)
    return pl.pallas_call(
        _proj_body,
        grid=grid,
        in_specs=[pl.BlockSpec((3, TC_BLK), lambda i: (0, i))],
        out_specs=[
            pl.BlockSpec((TC_BLK,), lambda i: (i,)),
            pl.BlockSpec((TC_BLK,), lambda i: (i,)),
        ],
        out_shape=[
            jax.ShapeDtypeStruct((B * N,), jnp.int32),
            jax.ShapeDtypeStruct((B * N,), jnp.float32),
        ],
    )(pts_t)


def _sc_vis_body(pix_hbm, d_hbm, out_hbm, pixc, dc, table, thresh, visc,
                 shared, psem0, psem1, dsem0, dsem1, osem0, osem1):
    c = lax.axis_index("c")
    s = lax.axis_index("s")
    b = c * 4 + s // 4
    r = s % 4
    rlo = r * NRANGE
    bslot = s // 4
    inf = jnp.float32(jnp.inf)
    psem = (psem0, psem1)
    dsem = (dsem0, dsem1)
    osem = (osem0, osem1)

    def _start_in(base, slot):
        pltpu.async_copy(pix_hbm.at[b, pl.ds(base, CHUNK)],
                         pixc.at[pl.ds(slot * CHUNK, CHUNK)], psem[slot])
        pltpu.async_copy(d_hbm.at[b, pl.ds(base, CHUNK)],
                         dc.at[pl.ds(slot * CHUNK, CHUNK)], dsem[slot])

    def _wait_in(base, slot):
        pltpu.make_async_copy(pix_hbm.at[b, pl.ds(base, CHUNK)],
                              pixc.at[pl.ds(slot * CHUNK, CHUNK)],
                              psem[slot]).wait()
        pltpu.make_async_copy(d_hbm.at[b, pl.ds(base, CHUNK)],
                              dc.at[pl.ds(slot * CHUNK, CHUNK)],
                              dsem[slot]).wait()

    _start_in(0, 0)

    def _init(i, carry):
        table[pl.ds(i * 16, 16)] = jnp.full((16,), inf, jnp.float32)
        return carry

    lax.fori_loop(0, 5 * NRANGE // 16, _init, 0)

    NCH1 = N // CHUNK

    def _chunk1(hi, carry):
      for slot in (0, 1):
        ci = hi * 2 + slot

        @pl.when(ci + 1 < NCH1)
        def _():
            _start_in((ci + 1) * CHUNK, 1 - slot)

        _wait_in(ci * CHUNK, slot)

        def _vec(v, carry2):
            p = pixc[pl.ds(slot * CHUNK + v * 16, 16)]
            dv = dc[pl.ds(slot * CHUNK + v * 16, 16)]
            u = p - rlo
            act0 = (u >= 0) & (u < NRANGE)
            us = jnp.where(act0, u, 0)
            occ = plsc.scan_count(us, mask=act0)[0].astype(jnp.int32)
            nmax = jnp.max(jnp.where(act0, occ, -1))

            def _round(i, carry3):
                sel = act0 & (occ == i)
                m0 = plsc.load_gather(table, [us])
                m1 = plsc.load_gather(table, [us + NRANGE])
                m2 = plsc.load_gather(table, [us + 2 * NRANGE])
                m3 = plsc.load_gather(table, [us + 3 * NRANGE])
                m4 = plsc.load_gather(table, [us + 4 * NRANGE])
                dup = ((dv == m0) | (dv == m1) | (dv == m2) | (dv == m3)
                       | (dv == m4))
                ins = sel & (~dup) & (dv < m4)
                t = dv
                n0 = jnp.minimum(m0, t)
                t = jnp.maximum(m0, t)
                n1 = jnp.minimum(m1, t)
                t = jnp.maximum(m1, t)
                n2 = jnp.minimum(m2, t)
                t = jnp.maximum(m2, t)
                n3 = jnp.minimum(m3, t)
                t = jnp.maximum(m3, t)
                n4 = jnp.minimum(m4, t)
                plsc.store_scatter(table, [us], n0, mask=ins)
                plsc.store_scatter(table, [us + NRANGE], n1, mask=ins)
                plsc.store_scatter(table, [us + 2 * NRANGE], n2, mask=ins)
                plsc.store_scatter(table, [us + 3 * NRANGE], n3, mask=ins)
                plsc.store_scatter(table, [us + 4 * NRANGE], n4, mask=ins)
                return carry3

            lax.fori_loop(0, nmax + 1, _round, 0)
            return carry2

        lax.fori_loop(0, CHUNK // 16, _vec, 0)
      return carry

    lax.fori_loop(0, NCH1 // 2, _chunk1, 0)

    plsc.subcore_barrier()
    pltpu.sync_copy(table.at[pl.ds(4 * NRANGE, NRANGE)],
                    shared.at[pl.ds(bslot * NSEG + rlo, NRANGE)])
    plsc.subcore_barrier()
    pltpu.sync_copy(shared.at[pl.ds(bslot * NSEG, NSEG)],
                    thresh.at[pl.ds(0, NSEG)])
    thresh[pl.ds(NSEG, 16)] = jnp.full((16,), -inf, jnp.float32)

    NCH3 = (N // 4) // CHUNK
    base0 = r * (N // 4)

    def _start_out(base, slot):
        pltpu.async_copy(visc.at[pl.ds(slot * CHUNK, CHUNK)],
                         out_hbm.at[b, pl.ds(base, CHUNK)], osem[slot])

    def _wait_out(base, slot):
        pltpu.make_async_copy(visc.at[pl.ds(slot * CHUNK, CHUNK)],
                              out_hbm.at[b, pl.ds(base, CHUNK)],
                              osem[slot]).wait()

    _start_in(base0, 0)

    def _chunk3(hi, carry):
      for slot in (0, 1):
        ci = hi * 2 + slot
        base = base0 + ci * CHUNK

        @pl.when(ci + 1 < NCH3)
        def _():
            _start_in(base + CHUNK, 1 - slot)

        _wait_in(base, slot)

        @pl.when(ci >= 2)
        def _():
            _wait_out(base - 2 * CHUNK, slot)

        def _vec(v, carry2):
            p = pixc[pl.ds(slot * CHUNK + v * 16, 16)]
            dv = dc[pl.ds(slot * CHUNK + v * 16, 16)]
            thr = plsc.load_gather(thresh, [p])
            vis = jnp.where(dv <= thr, jnp.float32(1.0), jnp.float32(0.0))
            visc[pl.ds(slot * CHUNK + v * 16, 16)] = vis
            return carry2

        lax.fori_loop(0, CHUNK // 16, _vec, 0)
        _start_out(base, slot)
      return carry

    lax.fori_loop(0, NCH3 // 2, _chunk3, 0)
    _wait_out(base0 + (NCH3 - 2) * CHUNK, 0)
    _wait_out(base0 + (NCH3 - 1) * CHUNK, 1)


_sc_vis = functools.partial(
    pl.kernel,
    mesh=plsc.VectorSubcoreMesh(core_axis_name="c", subcore_axis_name="s"),
    out_type=jax.ShapeDtypeStruct((B, N), jnp.float32),
    compiler_params=pltpu.CompilerParams(needs_layout_passes=False),
    scratch_types=[
        pltpu.VMEM((2 * CHUNK,), jnp.int32),
        pltpu.VMEM((2 * CHUNK,), jnp.float32),
        pltpu.VMEM((5 * NRANGE,), jnp.float32),
        pltpu.VMEM((NSEG + 16,), jnp.float32),
        pltpu.VMEM((2 * CHUNK,), jnp.float32),
        pltpu.VMEM_SHARED((4 * NSEG,), jnp.float32),
        pltpu.SemaphoreType.DMA,
        pltpu.SemaphoreType.DMA,
        pltpu.SemaphoreType.DMA,
        pltpu.SemaphoreType.DMA,
        pltpu.SemaphoreType.DMA,
        pltpu.SemaphoreType.DMA,
    ],
)(_sc_vis_body)


def kernel(pts):
    pts_t = jnp.transpose(pts, (2, 0, 1)).reshape(3, B * N)
    pix, d = _project(pts_t)
    return _sc_vis(pix.reshape(B, N), d.reshape(B, N))


# speedup vs baseline: 274.1604x; 1.9893x over previous
import functools

import jax
import jax.numpy as jnp
from jax import lax
from jax.experimental import pallas as pl
from jax.experimental.pallas import tpu as pltpu
from jax.experimental.pallas import tpu_sc as plsc

B = 8
N = 262144
IMG = 200
CAM_H = 0.65
NSEG = IMG * IMG
NRANGE = NSEG // 4
CHUNK = 2048
TC_BLK = 65536


def _proj_body(t_ref, pix_ref, d_ref):
    f = 1.0 / jnp.tan(jnp.deg2rad(jnp.float32(45.0)) / 2.0)
    x = t_ref[0]
    y = t_ref[1]
    z = t_ref[2]
    depth = CAM_H - y
    dok = depth > 1e-4
    safe = jnp.where(dok, depth, 1.0)
    ndc_x = x * f / safe
    ndc_y = z * f / safe
    px = jnp.floor((ndc_x + 1.0) * 0.5 * IMG).astype(jnp.int32)
    py = jnp.floor((ndc_y + 1.0) * 0.5 * IMG).astype(jnp.int32)
    valid = dok & (px >= 0) & (px < IMG) & (py >= 0) & (py < IMG)
    pix_ref[...] = jnp.where(valid, py * IMG + px, NSEG)
    d_ref[...] = jnp.where(valid, depth, jnp.inf).astype(jnp.float32)


def _project(pts_t):
    grid = (B * N // TC_BLK,)
    return pl.pallas_call(
        _proj_body,
        grid=grid,
        in_specs=[pl.BlockSpec((3, TC_BLK), lambda i: (0, i))],
        out_specs=[
            pl.BlockSpec((TC_BLK,), lambda i: (i,)),
            pl.BlockSpec((TC_BLK,), lambda i: (i,)),
        ],
        out_shape=[
            jax.ShapeDtypeStruct((B * N,), jnp.int32),
            jax.ShapeDtypeStruct((B * N,), jnp.float32),
        ],
    )(pts_t)


def _sc_vis_body(pix_hbm, d_hbm, out_hbm, pixc, dc, table, tag, thresh, visc,
                 shared, psem0, psem1, dsem0, dsem1, osem0, osem1):
    c = lax.axis_index("c")
    s = lax.axis_index("s")
    b = c * 4 + s // 4
    r = s % 4
    rlo = r * NRANGE
    bslot = s // 4
    inf = jnp.float32(jnp.inf)
    lane = lax.iota(jnp.int32, 16)

    def _insert(mask, us, dv):
        m0 = plsc.load_gather(table, [us])
        m1 = plsc.load_gather(table, [us + NRANGE])
        m2 = plsc.load_gather(table, [us + 2 * NRANGE])
        m3 = plsc.load_gather(table, [us + 3 * NRANGE])
        m4 = plsc.load_gather(table, [us + 4 * NRANGE])
        dup = ((dv == m0) | (dv == m1) | (dv == m2) | (dv == m3)
               | (dv == m4))
        ins = mask & (~dup) & (dv < m4)
        t = dv
        n0 = jnp.minimum(m0, t)
        t = jnp.maximum(m0, t)
        n1 = jnp.minimum(m1, t)
        t = jnp.maximum(m1, t)
        n2 = jnp.minimum(m2, t)
        t = jnp.maximum(m2, t)
        n3 = jnp.minimum(m3, t)
        t = jnp.maximum(m3, t)
        n4 = jnp.minimum(m4, t)
        plsc.store_scatter(table, [us], n0, mask=ins)
        plsc.store_scatter(table, [us + NRANGE], n1, mask=ins)
        plsc.store_scatter(table, [us + 2 * NRANGE], n2, mask=ins)
        plsc.store_scatter(table, [us + 3 * NRANGE], n3, mask=ins)
        plsc.store_scatter(table, [us + 4 * NRANGE], n4, mask=ins)
    psem = (psem0, psem1)
    dsem = (dsem0, dsem1)
    osem = (osem0, osem1)

    def _start_in(base, slot):
        pltpu.async_copy(pix_hbm.at[b, pl.ds(base, CHUNK)],
                         pixc.at[pl.ds(slot * CHUNK, CHUNK)], psem[slot])
        pltpu.async_copy(d_hbm.at[b, pl.ds(base, CHUNK)],
                         dc.at[pl.ds(slot * CHUNK, CHUNK)], dsem[slot])

    def _wait_in(base, slot):
        pltpu.make_async_copy(pix_hbm.at[b, pl.ds(base, CHUNK)],
                              pixc.at[pl.ds(slot * CHUNK, CHUNK)],
                              psem[slot]).wait()
        pltpu.make_async_copy(d_hbm.at[b, pl.ds(base, CHUNK)],
                              dc.at[pl.ds(slot * CHUNK, CHUNK)],
                              dsem[slot]).wait()

    _start_in(0, 0)

    def _init(i, carry):
        table[pl.ds(i * 16, 16)] = jnp.full((16,), inf, jnp.float32)
        return carry

    lax.fori_loop(0, 5 * NRANGE // 16, _init, 0)

    NCH1 = N // CHUNK

    def _chunk1(hi, carry):
      for slot in (0, 1):
        ci = hi * 2 + slot

        @pl.when(ci + 1 < NCH1)
        def _():
            _start_in((ci + 1) * CHUNK, 1 - slot)

        _wait_in(ci * CHUNK, slot)

        def _vec(v, carry2):
            p = pixc[pl.ds(slot * CHUNK + v * 16, 16)]
            dv = dc[pl.ds(slot * CHUNK + v * 16, 16)]
            u = p - rlo
            act0 = u.astype(jnp.uint32) < jnp.uint32(NRANGE)
            us = jnp.where(act0, u, 0)
            plsc.store_scatter(tag, [us], lane, mask=act0)
            got = plsc.load_gather(tag, [us])
            win = act0 & (got == lane)
            _insert(win, us, dv)
            rem = act0 & (~win)
            nrem = plsc.all_reduce_population_count(rem)[0]

            @pl.when(nrem > 0)
            def _():
                def _retry(i, rcarry):
                    plsc.store_scatter(tag, [us], lane, mask=rcarry)
                    g2 = plsc.load_gather(tag, [us])
                    w2 = rcarry & (g2 == lane)
                    _insert(w2, us, dv)
                    return rcarry & (~w2)

                lax.fori_loop(0, nrem, _retry, rem)

            return carry2

        lax.fori_loop(0, CHUNK // 16, _vec, 0, unroll=2)
      return carry

    lax.fori_loop(0, NCH1 // 2, _chunk1, 0)

    plsc.subcore_barrier()
    pltpu.sync_copy(table.at[pl.ds(4 * NRANGE, NRANGE)],
                    shared.at[pl.ds(bslot * NSEG + rlo, NRANGE)])
    plsc.subcore_barrier()
    pltpu.sync_copy(shared.at[pl.ds(bslot * NSEG, NSEG)],
                    thresh.at[pl.ds(0, NSEG)])
    thresh[pl.ds(NSEG, 16)] = jnp.full((16,), -inf, jnp.float32)

    NCH3 = (N // 4) // CHUNK
    base0 = r * (N // 4)

    def _start_out(base, slot):
        pltpu.async_copy(visc.at[pl.ds(slot * CHUNK, CHUNK)],
                         out_hbm.at[b, pl.ds(base, CHUNK)], osem[slot])

    def _wait_out(base, slot):
        pltpu.make_async_copy(visc.at[pl.ds(slot * CHUNK, CHUNK)],
                              out_hbm.at[b, pl.ds(base, CHUNK)],
                              osem[slot]).wait()

    _start_in(base0, 0)

    def _chunk3(hi, carry):
      for slot in (0, 1):
        ci = hi * 2 + slot
        base = base0 + ci * CHUNK

        @pl.when(ci + 1 < NCH3)
        def _():
            _start_in(base + CHUNK, 1 - slot)

        _wait_in(base, slot)

        @pl.when(ci >= 2)
        def _():
            _wait_out(base - 2 * CHUNK, slot)

        def _vec(v, carry2):
            p = pixc[pl.ds(slot * CHUNK + v * 16, 16)]
            dv = dc[pl.ds(slot * CHUNK + v * 16, 16)]
            thr = plsc.load_gather(thresh, [p])
            vis = jnp.where(dv <= thr, jnp.float32(1.0), jnp.float32(0.0))
            visc[pl.ds(slot * CHUNK + v * 16, 16)] = vis
            return carry2

        lax.fori_loop(0, CHUNK // 16, _vec, 0, unroll=4)
        _start_out(base, slot)
      return carry

    lax.fori_loop(0, NCH3 // 2, _chunk3, 0)
    _wait_out(base0 + (NCH3 - 2) * CHUNK, 0)
    _wait_out(base0 + (NCH3 - 1) * CHUNK, 1)


_sc_vis = functools.partial(
    pl.kernel,
    mesh=plsc.VectorSubcoreMesh(core_axis_name="c", subcore_axis_name="s"),
    out_type=jax.ShapeDtypeStruct((B, N), jnp.float32),
    compiler_params=pltpu.CompilerParams(needs_layout_passes=False),
    scratch_types=[
        pltpu.VMEM((2 * CHUNK,), jnp.int32),
        pltpu.VMEM((2 * CHUNK,), jnp.float32),
        pltpu.VMEM((5 * NRANGE,), jnp.float32),
        pltpu.VMEM((NRANGE,), jnp.int32),
        pltpu.VMEM((NSEG + 16,), jnp.float32),
        pltpu.VMEM((2 * CHUNK,), jnp.float32),
        pltpu.VMEM_SHARED((4 * NSEG,), jnp.float32),
        pltpu.SemaphoreType.DMA,
        pltpu.SemaphoreType.DMA,
        pltpu.SemaphoreType.DMA,
        pltpu.SemaphoreType.DMA,
        pltpu.SemaphoreType.DMA,
        pltpu.SemaphoreType.DMA,
    ],
)(_sc_vis_body)


def kernel(pts):
    pts_t = jnp.transpose(pts, (2, 0, 1)).reshape(3, B * N)
    pix, d = _project(pts_t)
    return _sc_vis(pix.reshape(B, N), d.reshape(B, N))


# speedup vs baseline: 371.2611x; 1.3542x over previous
import functools

import jax
import jax.numpy as jnp
from jax import lax
from jax.experimental import pallas as pl
from jax.experimental.pallas import tpu as pltpu
from jax.experimental.pallas import tpu_sc as plsc

B = 8
N = 262144
IMG = 200
CAM_H = 0.65
NSEG = IMG * IMG
NRANGE = NSEG // 4
CHUNK = 2048
TC_BLK = 65536


def _proj_body(t_ref, pix_ref, d_ref):
    f = 1.0 / jnp.tan(jnp.deg2rad(jnp.float32(45.0)) / 2.0)
    x = t_ref[0]
    y = t_ref[1]
    z = t_ref[2]
    depth = CAM_H - y
    dok = depth > 1e-4
    safe = jnp.where(dok, depth, 1.0)
    ndc_x = x * f / safe
    ndc_y = z * f / safe
    px = jnp.floor((ndc_x + 1.0) * 0.5 * IMG).astype(jnp.int32)
    py = jnp.floor((ndc_y + 1.0) * 0.5 * IMG).astype(jnp.int32)
    valid = dok & (px >= 0) & (px < IMG) & (py >= 0) & (py < IMG)
    pix_ref[...] = jnp.where(valid, py * IMG + px, NSEG)
    d_ref[...] = jnp.where(valid, depth, jnp.inf).astype(jnp.float32)


def _project(pts_t):
    grid = (B * N // TC_BLK,)
    return pl.pallas_call(
        _proj_body,
        grid=grid,
        in_specs=[pl.BlockSpec((3, TC_BLK), lambda i: (0, i))],
        out_specs=[
            pl.BlockSpec((TC_BLK,), lambda i: (i,)),
            pl.BlockSpec((TC_BLK,), lambda i: (i,)),
        ],
        out_shape=[
            jax.ShapeDtypeStruct((B * N,), jnp.int32),
            jax.ShapeDtypeStruct((B * N,), jnp.float32),
        ],
    )(pts_t)


def _sc_vis_body(pix_hbm, d_hbm, out_hbm, pixc, dc, cus, cdv, table, tag,
                 thresh, visc, shared, psem0, psem1, dsem0, dsem1, osem0,
                 osem1):
    c = lax.axis_index("c")
    s = lax.axis_index("s")
    b = c * 4 + s // 4
    r = s % 4
    rlo = r * NRANGE
    bslot = s // 4
    inf = jnp.float32(jnp.inf)
    lane = lax.iota(jnp.int32, 16)

    def _insert(mask, us, dv):
        m0 = plsc.load_gather(table, [us])
        m1 = plsc.load_gather(table, [us + NRANGE])
        m2 = plsc.load_gather(table, [us + 2 * NRANGE])
        m3 = plsc.load_gather(table, [us + 3 * NRANGE])
        m4 = plsc.load_gather(table, [us + 4 * NRANGE])
        dup = ((dv == m0) | (dv == m1) | (dv == m2) | (dv == m3)
               | (dv == m4))
        ins = mask & (~dup) & (dv < m4)
        t = dv
        n0 = jnp.minimum(m0, t)
        t = jnp.maximum(m0, t)
        n1 = jnp.minimum(m1, t)
        t = jnp.maximum(m1, t)
        n2 = jnp.minimum(m2, t)
        t = jnp.maximum(m2, t)
        n3 = jnp.minimum(m3, t)
        t = jnp.maximum(m3, t)
        n4 = jnp.minimum(m4, t)
        plsc.store_scatter(table, [us], n0, mask=ins)
        plsc.store_scatter(table, [us + NRANGE], n1, mask=ins)
        plsc.store_scatter(table, [us + 2 * NRANGE], n2, mask=ins)
        plsc.store_scatter(table, [us + 3 * NRANGE], n3, mask=ins)
        plsc.store_scatter(table, [us + 4 * NRANGE], n4, mask=ins)
    psem = (psem0, psem1)
    dsem = (dsem0, dsem1)
    osem = (osem0, osem1)

    def _start_in(base, slot):
        pltpu.async_copy(pix_hbm.at[b, pl.ds(base, CHUNK)],
                         pixc.at[pl.ds(slot * CHUNK, CHUNK)], psem[slot])
        pltpu.async_copy(d_hbm.at[b, pl.ds(base, CHUNK)],
                         dc.at[pl.ds(slot * CHUNK, CHUNK)], dsem[slot])

    def _wait_in(base, slot):
        pltpu.make_async_copy(pix_hbm.at[b, pl.ds(base, CHUNK)],
                              pixc.at[pl.ds(slot * CHUNK, CHUNK)],
                              psem[slot]).wait()
        pltpu.make_async_copy(d_hbm.at[b, pl.ds(base, CHUNK)],
                              dc.at[pl.ds(slot * CHUNK, CHUNK)],
                              dsem[slot]).wait()

    _start_in(0, 0)

    def _init(i, carry):
        table[pl.ds(i * 16, 16)] = jnp.full((16,), inf, jnp.float32)
        return carry

    lax.fori_loop(0, 5 * NRANGE // 16, _init, 0)

    NCH1 = N // CHUNK

    def _chunk1(hi, carry):
      for slot in (0, 1):
        ci = hi * 2 + slot

        @pl.when(ci + 1 < NCH1)
        def _():
            _start_in((ci + 1) * CHUNK, 1 - slot)

        _wait_in(ci * CHUNK, slot)

        def _scan(v, cnt):
            p = pixc[pl.ds(slot * CHUNK + v * 16, 16)]
            dv = dc[pl.ds(slot * CHUNK + v * 16, 16)]
            u = p - rlo
            act = u.astype(jnp.uint32) < jnp.uint32(NRANGE)
            plsc.store_compressed(cus.at[pl.ds(cnt, 16)], u, mask=act)
            plsc.store_compressed(cdv.at[pl.ds(cnt, 16)], dv, mask=act)
            return cnt + plsc.all_reduce_population_count(act)[0]

        total = lax.fori_loop(0, CHUNK // 16, _scan, jnp.int32(0), unroll=4)

        def _vec(w, carry2):
            us0 = cus[pl.ds(w * 16, 16)]
            dv = cdv[pl.ds(w * 16, 16)]
            act0 = lane < (total - w * 16)
            us = jnp.where(act0, us0, 0)
            plsc.store_scatter(tag, [us], lane, mask=act0)
            got = plsc.load_gather(tag, [us])
            win = act0 & (got == lane)
            _insert(win, us, dv)
            rem = act0 & (~win)
            nrem = plsc.all_reduce_population_count(rem)[0]

            @pl.when(nrem > 0)
            def _():
                def _retry(i, rcarry):
                    plsc.store_scatter(tag, [us], lane, mask=rcarry)
                    g2 = plsc.load_gather(tag, [us])
                    w2 = rcarry & (g2 == lane)
                    _insert(w2, us, dv)
                    return rcarry & (~w2)

                lax.fori_loop(0, nrem, _retry, rem)

            return carry2

        lax.fori_loop(0, (total + 15) // 16, _vec, 0)
      return carry

    lax.fori_loop(0, NCH1 // 2, _chunk1, 0)

    plsc.subcore_barrier()
    pltpu.sync_copy(table.at[pl.ds(4 * NRANGE, NRANGE)],
                    shared.at[pl.ds(bslot * NSEG + rlo, NRANGE)])
    plsc.subcore_barrier()
    pltpu.sync_copy(shared.at[pl.ds(bslot * NSEG, NSEG)],
                    thresh.at[pl.ds(0, NSEG)])
    thresh[pl.ds(NSEG, 16)] = jnp.full((16,), -inf, jnp.float32)

    NCH3 = (N // 4) // CHUNK
    base0 = r * (N // 4)

    def _start_out(base, slot):
        pltpu.async_copy(visc.at[pl.ds(slot * CHUNK, CHUNK)],
                         out_hbm.at[b, pl.ds(base, CHUNK)], osem[slot])

    def _wait_out(base, slot):
        pltpu.make_async_copy(visc.at[pl.ds(slot * CHUNK, CHUNK)],
                              out_hbm.at[b, pl.ds(base, CHUNK)],
                              osem[slot]).wait()

    _start_in(base0, 0)

    def _chunk3(hi, carry):
      for slot in (0, 1):
        ci = hi * 2 + slot
        base = base0 + ci * CHUNK

        @pl.when(ci + 1 < NCH3)
        def _():
            _start_in(base + CHUNK, 1 - slot)

        _wait_in(base, slot)

        @pl.when(ci >= 2)
        def _():
            _wait_out(base - 2 * CHUNK, slot)

        def _vec(v, carry2):
            p = pixc[pl.ds(slot * CHUNK + v * 16, 16)]
            dv = dc[pl.ds(slot * CHUNK + v * 16, 16)]
            thr = plsc.load_gather(thresh, [p])
            vis = jnp.where(dv <= thr, jnp.float32(1.0), jnp.float32(0.0))
            visc[pl.ds(slot * CHUNK + v * 16, 16)] = vis
            return carry2

        lax.fori_loop(0, CHUNK // 16, _vec, 0, unroll=4)
        _start_out(base, slot)
      return carry

    lax.fori_loop(0, NCH3 // 2, _chunk3, 0)
    _wait_out(base0 + (NCH3 - 2) * CHUNK, 0)
    _wait_out(base0 + (NCH3 - 1) * CHUNK, 1)


_sc_vis = functools.partial(
    pl.kernel,
    mesh=plsc.VectorSubcoreMesh(core_axis_name="c", subcore_axis_name="s"),
    out_type=jax.ShapeDtypeStruct((B, N), jnp.float32),
    compiler_params=pltpu.CompilerParams(needs_layout_passes=False),
    scratch_types=[
        pltpu.VMEM((2 * CHUNK,), jnp.int32),
        pltpu.VMEM((2 * CHUNK,), jnp.float32),
        pltpu.VMEM((CHUNK + 16,), jnp.int32),
        pltpu.VMEM((CHUNK + 16,), jnp.float32),
        pltpu.VMEM((5 * NRANGE,), jnp.float32),
        pltpu.VMEM((NRANGE,), jnp.int32),
        pltpu.VMEM((NSEG + 16,), jnp.float32),
        pltpu.VMEM((2 * CHUNK,), jnp.float32),
        pltpu.VMEM_SHARED((4 * NSEG,), jnp.float32),
        pltpu.SemaphoreType.DMA,
        pltpu.SemaphoreType.DMA,
        pltpu.SemaphoreType.DMA,
        pltpu.SemaphoreType.DMA,
        pltpu.SemaphoreType.DMA,
        pltpu.SemaphoreType.DMA,
    ],
)(_sc_vis_body)


def kernel(pts):
    pts_t = jnp.transpose(pts, (2, 0, 1)).reshape(3, B * N)
    pix, d = _project(pts_t)
    return _sc_vis(pix.reshape(B, N), d.reshape(B, N))
